# Initial kernel scaffold; baseline (speedup 1.0000x reference)
import functools

import jax
import jax.numpy as jnp
from jax import lax
from jax.experimental import pallas as pl
from jax.experimental.pallas import tpu as pltpu
from jax.experimental.pallas import tpu_sc as plsc

N = 10000
E = 640000
D_IN = 32
H = 128
OUT = 4

NC = 2
NS = 16
NW = NC * NS
CH = 128
NCH = 158
EPW = NCH * CH
E_PAD = NW * EPW
NP = 10240
ZR = NP // NS

B1 = 1280
G1 = NP // B1


def _tc1_body(f, w_in, b_in, wl1, wr1, wl2, wr2, wo, o_ref):
    x = jnp.dot(f[...], w_in[...], preferred_element_type=jnp.float32) + b_in[...]
    x0 = jnp.where(x >= 0, x, 0.01 * x)
    a = jnp.dot(wl2[...], wo[...], preferred_element_type=jnp.float32)
    b = jnp.dot(wr2[...], wo[...], preferred_element_type=jnp.float32)
    m1 = jnp.dot(wl1[...], a, preferred_element_type=jnp.float32)
    mq = (jnp.dot(wl1[...], b, preferred_element_type=jnp.float32)
          + jnp.dot(wr1[...], a, preferred_element_type=jnp.float32))
    m3 = jnp.dot(wr1[...], b, preferred_element_type=jnp.float32)
    p = jnp.dot(x0, m1, preferred_element_type=jnp.float32)
    q = jnp.dot(x0, mq, preferred_element_type=jnp.float32)
    r = jnp.dot(x0, m3, preferred_element_type=jnp.float32)
    ones = jnp.ones((B1, 1), jnp.float32)
    zeros3 = jnp.zeros((B1, 3), jnp.float32)
    o_ref[...] = jnp.concatenate([p, q, ones, r, zeros3], axis=1)


def _tc1(fpad, w_in, b_in2, wl1, wr1, wl2, wr2, wo):
    full = lambda s: pl.BlockSpec(s, lambda i: (0, 0))
    return pl.pallas_call(
        _tc1_body,
        grid=(G1,),
        in_specs=[
            pl.BlockSpec((B1, D_IN), lambda i: (i, 0)),
            full((D_IN, H)), full((1, H)), full((H, H)), full((H, H)),
            full((H, H)), full((H, H)), full((H, OUT)),
        ],
        out_specs=pl.BlockSpec((B1, 16), lambda i: (i, 0)),
        out_shape=jax.ShapeDtypeStruct((NP, 16), jnp.float32),
    )(fpad, w_in, b_in2, wl1, wr1, wl2, wr2, wo)


def _edge_body(vals_hbm, src_hbm, dst_hbm, zrows_hbm, out_hbm,
               shared_acc, srcv, dstv, rows, sem):
    c = lax.axis_index("c")
    s = lax.axis_index("s")
    wid = s * NC + c
    pltpu.sync_copy(zrows_hbm, shared_acc.at[pl.ds(s * ZR, ZR)])
    plsc.subcore_barrier()
    pltpu.sync_copy(src_hbm.at[wid], srcv)
    pltpu.sync_copy(dst_hbm.at[wid], dstv)

    def chunk(j, carry):
        pltpu.async_copy(vals_hbm.at[srcv.at[j]], rows, sem).wait()
        pltpu.sync_copy(rows, shared_acc.at[dstv.at[j]], add=True)
        return carry

    lax.fori_loop(0, NCH, chunk, 0)
    plsc.subcore_barrier()
    pltpu.sync_copy(shared_acc.at[pl.ds(s * ZR, ZR)],
                    out_hbm.at[c].at[pl.ds(s * ZR, ZR)])


def _edge_pass(vals16, src3, dst3, zrows):
    mesh = plsc.VectorSubcoreMesh(core_axis_name="c", subcore_axis_name="s")
    fn = pl.kernel(
        _edge_body,
        out_type=jax.ShapeDtypeStruct((NC, NP, 16), jnp.float32),
        mesh=mesh,
        scratch_types=[
            pltpu.VMEM_SHARED((NP, 16), jnp.float32),
            pltpu.VMEM((NCH, CH), jnp.int32),
            pltpu.VMEM((NCH, CH), jnp.int32),
            pltpu.VMEM((CH, 16), jnp.float32),
            pltpu.SemaphoreType.DMA,
        ],
    )
    return fn(vals16, src3, dst3, zrows)


def _comb_body(acc, pq, wl2, wr2, wo, bl1, bl2, bo, u_ref, s_ref):
    av = acc[...]
    sm = av[0] + av[1]
    cnt = sm[:, 8:9]
    dinv = 1.0 / jnp.maximum(cnt, 1.0)
    row = sm * dinv
    ones = jnp.ones((B1, 1), jnp.float32)
    u_ref[...] = jnp.concatenate(
        [row[:, 0:4], jnp.zeros((B1, 4), jnp.float32), ones,
         jnp.zeros((B1, 7), jnp.float32)], axis=1)
    a = jnp.dot(wl2[...], wo[...], preferred_element_type=jnp.float32)
    b = jnp.dot(wr2[...], wo[...], preferred_element_type=jnp.float32)
    c1 = jnp.dot(bl1[...], a, preferred_element_type=jnp.float32)
    c0 = (jnp.dot(bl1[...], b, preferred_element_type=jnp.float32)
          + jnp.dot(bl2[...], wo[...], preferred_element_type=jnp.float32)
          + bo[...])
    m = row[:, 8:9]
    s_ref[...] = row[:, 4:8] + m * c1 + pq[:, 9:13] + c0


def _combine(acc, pq16, wl2, wr2, wo, bl1_2, bl2_2, bo2):
    full = lambda s: pl.BlockSpec(s, lambda i: (0, 0))
    return pl.pallas_call(
        _comb_body,
        grid=(G1,),
        in_specs=[
            pl.BlockSpec((NC, B1, 16), lambda i: (0, i, 0)),
            pl.BlockSpec((B1, 16), lambda i: (i, 0)),
            full((H, H)), full((H, H)), full((H, OUT)),
            full((1, H)), full((1, H)), full((1, OUT)),
        ],
        out_specs=[
            pl.BlockSpec((B1, 16), lambda i: (i, 0)),
            pl.BlockSpec((B1, OUT), lambda i: (i, 0)),
        ],
        out_shape=[
            jax.ShapeDtypeStruct((NP, 16), jnp.float32),
            jax.ShapeDtypeStruct((NP, OUT), jnp.float32),
        ],
    )(acc, pq16, wl2, wr2, wo, bl1_2, bl2_2, bo2)


def _fin_body(acc, s4, o_ref):
    av = acc[...]
    sm = av[0] + av[1]
    cnt = sm[:, 8:9]
    dinv = 1.0 / jnp.maximum(cnt, 1.0)
    o_ref[...] = sm[:, 0:4] * dinv + s4[...]


def _final(acc2, s4):
    return pl.pallas_call(
        _fin_body,
        grid=(G1,),
        in_specs=[
            pl.BlockSpec((NC, B1, 16), lambda i: (0, i, 0)),
            pl.BlockSpec((B1, OUT), lambda i: (i, 0)),
        ],
        out_specs=pl.BlockSpec((B1, OUT), lambda i: (i, 0)),
        out_shape=jax.ShapeDtypeStruct((NP, OUT), jnp.float32),
    )(acc2, s4)


def kernel(feature, edge_index, edge_type, W_in, b_in, Wl1, bl1, Wr1,
           Wl2, bl2, Wr2, Wo, bo):
    del edge_type
    f32 = jnp.float32
    fpad = jnp.zeros((NP, D_IN), f32).at[:N].set(feature.astype(f32))
    src = edge_index[0]
    dst = edge_index[1]
    pad = jnp.full((E_PAD - E,), NP - 1, jnp.int32)
    src3 = jnp.concatenate([src, pad]).reshape(NW, NCH, CH)
    dst3 = jnp.concatenate([dst, pad]).reshape(NW, NCH, CH)
    zrows = jnp.zeros((ZR, 16), f32)

    b_in2 = b_in.reshape(1, H)
    bl1_2 = bl1.reshape(1, H)
    bl2_2 = bl2.reshape(1, H)
    bo2 = bo.reshape(1, OUT)

    pq16 = _tc1(fpad, W_in, b_in2, Wl1, Wr1, Wl2, Wr2, Wo)
    acc1 = _edge_pass(pq16, src3, dst3, zrows)
    u16, s4 = _combine(acc1, pq16, Wl2, Wr2, Wo, bl1_2, bl2_2, bo2)
    acc2 = _edge_pass(u16, src3, dst3, zrows)
    outp = _final(acc2, s4)
    return outp[:N]


# speedup vs baseline: 20.4877x; 20.4877x over previous
import functools

import jax
import jax.numpy as jnp
from jax import lax
from jax.experimental import pallas as pl
from jax.experimental.pallas import tpu as pltpu
from jax.experimental.pallas import tpu_sc as plsc

N = 10000
E = 640000
D_IN = 32
H = 128
OUT = 4

NC = 2
NS = 16
NW = NC * NS
CH = 128
NCH = 158
EPW = NCH * CH
E_PAD = NW * EPW
NP = 10240
ZR = NP // NS

B1 = 1280
G1 = NP // B1


def _tc1_body(f, w_in, b_in, wl1, wr1, wl2, wr2, wo, o_ref):
    x = jnp.dot(f[...], w_in[...], preferred_element_type=jnp.float32) + b_in[...]
    x0 = jnp.where(x >= 0, x, 0.01 * x)
    a = jnp.dot(wl2[...], wo[...], preferred_element_type=jnp.float32)
    b = jnp.dot(wr2[...], wo[...], preferred_element_type=jnp.float32)
    m1 = jnp.dot(wl1[...], a, preferred_element_type=jnp.float32)
    mq = (jnp.dot(wl1[...], b, preferred_element_type=jnp.float32)
          + jnp.dot(wr1[...], a, preferred_element_type=jnp.float32))
    m3 = jnp.dot(wr1[...], b, preferred_element_type=jnp.float32)
    p = jnp.dot(x0, m1, preferred_element_type=jnp.float32)
    q = jnp.dot(x0, mq, preferred_element_type=jnp.float32)
    r = jnp.dot(x0, m3, preferred_element_type=jnp.float32)
    ones = jnp.ones((B1, 1), jnp.float32)
    zeros3 = jnp.zeros((B1, 3), jnp.float32)
    o_ref[...] = jnp.concatenate([p, q, ones, r, zeros3], axis=1)


def _tc1(fpad, w_in, b_in2, wl1, wr1, wl2, wr2, wo):
    full = lambda s: pl.BlockSpec(s, lambda i: (0, 0))
    return pl.pallas_call(
        _tc1_body,
        grid=(G1,),
        in_specs=[
            pl.BlockSpec((B1, D_IN), lambda i: (i, 0)),
            full((D_IN, H)), full((1, H)), full((H, H)), full((H, H)),
            full((H, H)), full((H, H)), full((H, OUT)),
        ],
        out_specs=pl.BlockSpec((B1, 16), lambda i: (i, 0)),
        out_shape=jax.ShapeDtypeStruct((NP, 16), jnp.float32),
    )(fpad, w_in, b_in2, wl1, wr1, wl2, wr2, wo)


def _edge_body(vals_hbm, src_hbm, dst_hbm, zrows_hbm, out_hbm,
               shared_acc, srcv, dstv, rows, sem):
    c = lax.axis_index("c")
    s = lax.axis_index("s")
    wid = s * NC + c
    pltpu.sync_copy(zrows_hbm, shared_acc.at[pl.ds(s * ZR, ZR)])
    plsc.subcore_barrier()
    pltpu.sync_copy(src_hbm.at[wid], srcv)
    pltpu.sync_copy(dst_hbm.at[wid], dstv)

    def chunk(j, carry):
        pltpu.async_copy(vals_hbm.at[srcv.at[j]], rows, sem).wait()
        pltpu.sync_copy(rows, shared_acc.at[dstv.at[j]], add=True)
        return carry

    lax.fori_loop(0, NCH, chunk, 0)
    plsc.subcore_barrier()
    pltpu.sync_copy(shared_acc.at[pl.ds(s * ZR, ZR)],
                    out_hbm.at[c].at[pl.ds(s * ZR, ZR)])


def _edge_pass(vals16, src3, dst3, zrows):
    mesh = plsc.VectorSubcoreMesh(core_axis_name="c", subcore_axis_name="s")
    fn = pl.kernel(
        _edge_body,
        out_type=jax.ShapeDtypeStruct((NC, NP, 16), jnp.float32),
        mesh=mesh,
        scratch_types=[
            pltpu.VMEM_SHARED((NP, 16), jnp.float32),
            pltpu.VMEM((NCH, CH), jnp.int32),
            pltpu.VMEM((NCH, CH), jnp.int32),
            pltpu.VMEM((CH, 16), jnp.float32),
            pltpu.SemaphoreType.DMA,
        ],
        compiler_params=pltpu.CompilerParams(use_tc_tiling_on_sc=False),
    )
    return fn(vals16, src3, dst3, zrows)


def _comb_body(acc, pq, wl2, wr2, wo, bl1, bl2, bo, u_ref, s_ref):
    av = acc[...]
    sm = av[0] + av[1]
    cnt = sm[:, 8:9]
    dinv = 1.0 / jnp.maximum(cnt, 1.0)
    row = sm * dinv
    ones = jnp.ones((B1, 1), jnp.float32)
    u_ref[...] = jnp.concatenate(
        [row[:, 0:4], jnp.zeros((B1, 4), jnp.float32), ones,
         jnp.zeros((B1, 7), jnp.float32)], axis=1)
    a = jnp.dot(wl2[...], wo[...], preferred_element_type=jnp.float32)
    b = jnp.dot(wr2[...], wo[...], preferred_element_type=jnp.float32)
    c1 = jnp.dot(bl1[...], a, preferred_element_type=jnp.float32)
    c0 = (jnp.dot(bl1[...], b, preferred_element_type=jnp.float32)
          + jnp.dot(bl2[...], wo[...], preferred_element_type=jnp.float32)
          + bo[...])
    m = row[:, 8:9]
    s_ref[...] = row[:, 4:8] + m * c1 + pq[:, 9:13] + c0


def _combine(acc, pq16, wl2, wr2, wo, bl1_2, bl2_2, bo2):
    full = lambda s: pl.BlockSpec(s, lambda i: (0, 0))
    return pl.pallas_call(
        _comb_body,
        grid=(G1,),
        in_specs=[
            pl.BlockSpec((NC, B1, 16), lambda i: (0, i, 0)),
            pl.BlockSpec((B1, 16), lambda i: (i, 0)),
            full((H, H)), full((H, H)), full((H, OUT)),
            full((1, H)), full((1, H)), full((1, OUT)),
        ],
        out_specs=[
            pl.BlockSpec((B1, 16), lambda i: (i, 0)),
            pl.BlockSpec((B1, OUT), lambda i: (i, 0)),
        ],
        out_shape=[
            jax.ShapeDtypeStruct((NP, 16), jnp.float32),
            jax.ShapeDtypeStruct((NP, OUT), jnp.float32),
        ],
    )(acc, pq16, wl2, wr2, wo, bl1_2, bl2_2, bo2)


def _fin_body(acc, s4, o_ref):
    av = acc[...]
    sm = av[0] + av[1]
    cnt = sm[:, 8:9]
    dinv = 1.0 / jnp.maximum(cnt, 1.0)
    o_ref[...] = sm[:, 0:4] * dinv + s4[...]


def _final(acc2, s4):
    return pl.pallas_call(
        _fin_body,
        grid=(G1,),
        in_specs=[
            pl.BlockSpec((NC, B1, 16), lambda i: (0, i, 0)),
            pl.BlockSpec((B1, OUT), lambda i: (i, 0)),
        ],
        out_specs=pl.BlockSpec((B1, OUT), lambda i: (i, 0)),
        out_shape=jax.ShapeDtypeStruct((NP, OUT), jnp.float32),
    )(acc2, s4)


def kernel(feature, edge_index, edge_type, W_in, b_in, Wl1, bl1, Wr1,
           Wl2, bl2, Wr2, Wo, bo):
    del edge_type
    f32 = jnp.float32
    fpad = jnp.zeros((NP, D_IN), f32).at[:N].set(feature.astype(f32))
    src = edge_index[0]
    dst = edge_index[1]
    pad = jnp.full((E_PAD - E,), NP - 1, jnp.int32)
    src3 = jnp.concatenate([src, pad]).reshape(NW, NCH, CH)
    dst3 = jnp.concatenate([dst, pad]).reshape(NW, NCH, CH)
    zrows = jnp.zeros((ZR, 16), f32)

    b_in2 = b_in.reshape(1, H)
    bl1_2 = bl1.reshape(1, H)
    bl2_2 = bl2.reshape(1, H)
    bo2 = bo.reshape(1, OUT)

    pq16 = _tc1(fpad, W_in, b_in2, Wl1, Wr1, Wl2, Wr2, Wo)
    acc1 = _edge_pass(pq16, src3, dst3, zrows)
    u16, s4 = _combine(acc1, pq16, Wl2, Wr2, Wo, bl1_2, bl2_2, bo2)
    acc2 = _edge_pass(u16, src3, dst3, zrows)
    outp = _final(acc2, s4)
    return outp[:N]


# speedup vs baseline: 24.6140x; 1.2014x over previous
import functools

import jax
import jax.numpy as jnp
from jax import lax
from jax.experimental import pallas as pl
from jax.experimental.pallas import tpu as pltpu
from jax.experimental.pallas import tpu_sc as plsc

N = 10000
E = 640000
D_IN = 32
H = 128
OUT = 4

NC = 2
NS = 16
NW = NC * NS
CH = 128
NBUF = 8
NCH = 160
NG = NCH // NBUF
EPW = NCH * CH
E_PAD = NW * EPW
NP = 10240
ZR = NP // NS

B1 = 1280
G1 = NP // B1


def _tc1_body(f, w_in, b_in, wl1, wr1, wl2, wr2, wo, o_ref):
    x = jnp.dot(f[...], w_in[...], preferred_element_type=jnp.float32) + b_in[...]
    x0 = jnp.where(x >= 0, x, 0.01 * x)
    a = jnp.dot(wl2[...], wo[...], preferred_element_type=jnp.float32)
    b = jnp.dot(wr2[...], wo[...], preferred_element_type=jnp.float32)
    m1 = jnp.dot(wl1[...], a, preferred_element_type=jnp.float32)
    mq = (jnp.dot(wl1[...], b, preferred_element_type=jnp.float32)
          + jnp.dot(wr1[...], a, preferred_element_type=jnp.float32))
    m3 = jnp.dot(wr1[...], b, preferred_element_type=jnp.float32)
    p = jnp.dot(x0, m1, preferred_element_type=jnp.float32)
    q = jnp.dot(x0, mq, preferred_element_type=jnp.float32)
    r = jnp.dot(x0, m3, preferred_element_type=jnp.float32)
    ones = jnp.ones((B1, 1), jnp.float32)
    zeros3 = jnp.zeros((B1, 3), jnp.float32)
    o_ref[...] = jnp.concatenate([p, q, ones, r, zeros3], axis=1)


def _tc1(fpad, w_in, b_in2, wl1, wr1, wl2, wr2, wo):
    full = lambda s: pl.BlockSpec(s, lambda i: (0, 0))
    return pl.pallas_call(
        _tc1_body,
        grid=(G1,),
        in_specs=[
            pl.BlockSpec((B1, D_IN), lambda i: (i, 0)),
            full((D_IN, H)), full((1, H)), full((H, H)), full((H, H)),
            full((H, H)), full((H, H)), full((H, OUT)),
        ],
        out_specs=pl.BlockSpec((B1, 16), lambda i: (i, 0)),
        out_shape=jax.ShapeDtypeStruct((NP, 16), jnp.float32),
    )(fpad, w_in, b_in2, wl1, wr1, wl2, wr2, wo)


def _edge_body(vals_hbm, src_hbm, dst_hbm, zrows_hbm, out_hbm,
               shared_acc, srcv, dstv, rows, sem, ssem):
    c = lax.axis_index("c")
    s = lax.axis_index("s")
    wid = s * NC + c
    pltpu.sync_copy(zrows_hbm, shared_acc.at[pl.ds(s * ZR, ZR)])
    plsc.subcore_barrier()
    pltpu.sync_copy(src_hbm.at[wid], srcv)
    pltpu.sync_copy(dst_hbm.at[wid], dstv)

    def group(g, carry):
        j0 = g * NBUF
        gathers = [
            pltpu.async_copy(vals_hbm.at[srcv.at[j0 + b]], rows.at[b], sem)
            for b in range(NBUF)
        ]
        for d in gathers:
            d.wait()
        scatters = [
            pltpu.async_copy(rows.at[b], shared_acc.at[dstv.at[j0 + b]],
                             ssem, add=True)
            for b in range(NBUF)
        ]
        for d in scatters:
            d.wait()
        return carry

    lax.fori_loop(0, NG, group, 0)
    plsc.subcore_barrier()
    pltpu.sync_copy(shared_acc.at[pl.ds(s * ZR, ZR)],
                    out_hbm.at[c].at[pl.ds(s * ZR, ZR)])


def _edge_pass(vals16, src3, dst3, zrows):
    mesh = plsc.VectorSubcoreMesh(core_axis_name="c", subcore_axis_name="s")
    fn = pl.kernel(
        _edge_body,
        out_type=jax.ShapeDtypeStruct((NC, NP, 16), jnp.float32),
        mesh=mesh,
        scratch_types=[
            pltpu.VMEM_SHARED((NP, 16), jnp.float32),
            pltpu.VMEM((NCH, CH), jnp.int32),
            pltpu.VMEM((NCH, CH), jnp.int32),
            pltpu.VMEM((NBUF, CH, 16), jnp.float32),
            pltpu.SemaphoreType.DMA,
            pltpu.SemaphoreType.DMA,
        ],
        compiler_params=pltpu.CompilerParams(use_tc_tiling_on_sc=False),
    )
    return fn(vals16, src3, dst3, zrows)


def _comb_body(acc, pq, wl2, wr2, wo, bl1, bl2, bo, u_ref, s_ref):
    av = acc[...]
    sm = av[0] + av[1]
    cnt = sm[:, 8:9]
    dinv = 1.0 / jnp.maximum(cnt, 1.0)
    row = sm * dinv
    ones = jnp.ones((B1, 1), jnp.float32)
    u_ref[...] = jnp.concatenate(
        [row[:, 0:4], jnp.zeros((B1, 4), jnp.float32), ones,
         jnp.zeros((B1, 7), jnp.float32)], axis=1)
    a = jnp.dot(wl2[...], wo[...], preferred_element_type=jnp.float32)
    b = jnp.dot(wr2[...], wo[...], preferred_element_type=jnp.float32)
    c1 = jnp.dot(bl1[...], a, preferred_element_type=jnp.float32)
    c0 = (jnp.dot(bl1[...], b, preferred_element_type=jnp.float32)
          + jnp.dot(bl2[...], wo[...], preferred_element_type=jnp.float32)
          + bo[...])
    m = row[:, 8:9]
    s_ref[...] = row[:, 4:8] + m * c1 + pq[:, 9:13] + c0


def _combine(acc, pq16, wl2, wr2, wo, bl1_2, bl2_2, bo2):
    full = lambda s: pl.BlockSpec(s, lambda i: (0, 0))
    return pl.pallas_call(
        _comb_body,
        grid=(G1,),
        in_specs=[
            pl.BlockSpec((NC, B1, 16), lambda i: (0, i, 0)),
            pl.BlockSpec((B1, 16), lambda i: (i, 0)),
            full((H, H)), full((H, H)), full((H, OUT)),
            full((1, H)), full((1, H)), full((1, OUT)),
        ],
        out_specs=[
            pl.BlockSpec((B1, 16), lambda i: (i, 0)),
            pl.BlockSpec((B1, OUT), lambda i: (i, 0)),
        ],
        out_shape=[
            jax.ShapeDtypeStruct((NP, 16), jnp.float32),
            jax.ShapeDtypeStruct((NP, OUT), jnp.float32),
        ],
    )(acc, pq16, wl2, wr2, wo, bl1_2, bl2_2, bo2)


def _fin_body(acc, s4, o_ref):
    av = acc[...]
    sm = av[0] + av[1]
    cnt = sm[:, 8:9]
    dinv = 1.0 / jnp.maximum(cnt, 1.0)
    o_ref[...] = sm[:, 0:4] * dinv + s4[...]


def _final(acc2, s4):
    return pl.pallas_call(
        _fin_body,
        grid=(G1,),
        in_specs=[
            pl.BlockSpec((NC, B1, 16), lambda i: (0, i, 0)),
            pl.BlockSpec((B1, OUT), lambda i: (i, 0)),
        ],
        out_specs=pl.BlockSpec((B1, OUT), lambda i: (i, 0)),
        out_shape=jax.ShapeDtypeStruct((NP, OUT), jnp.float32),
    )(acc2, s4)


def kernel(feature, edge_index, edge_type, W_in, b_in, Wl1, bl1, Wr1,
           Wl2, bl2, Wr2, Wo, bo):
    del edge_type
    f32 = jnp.float32
    fpad = jnp.zeros((NP, D_IN), f32).at[:N].set(feature.astype(f32))
    src = edge_index[0]
    dst = edge_index[1]
    pad = jnp.full((E_PAD - E,), NP - 1, jnp.int32)
    src3 = jnp.concatenate([src, pad]).reshape(NW, NCH, CH)
    dst3 = jnp.concatenate([dst, pad]).reshape(NW, NCH, CH)
    zrows = jnp.zeros((ZR, 16), f32)

    b_in2 = b_in.reshape(1, H)
    bl1_2 = bl1.reshape(1, H)
    bl2_2 = bl2.reshape(1, H)
    bo2 = bo.reshape(1, OUT)

    pq16 = _tc1(fpad, W_in, b_in2, Wl1, Wr1, Wl2, Wr2, Wo)
    acc1 = _edge_pass(pq16, src3, dst3, zrows)
    u16, s4 = _combine(acc1, pq16, Wl2, Wr2, Wo, bl1_2, bl2_2, bo2)
    acc2 = _edge_pass(u16, src3, dst3, zrows)
    outp = _final(acc2, s4)
    return outp[:N]
